# baseline (speedup 1.0000x reference)
import functools

import jax
import jax.numpy as jnp
from jax import lax
from jax.experimental import pallas as pl
from jax.experimental.pallas import tpu as pltpu
from jax.experimental.pallas import tpu_sc as plsc

VOCAB = 168960
HIDDEN = 4096
N_TOK = 4 * 8192

NC = 2
NS = 16
NW = NC * NS
B_PER_W = N_TOK // NW
CA = 16
CB = 8
PAIR = CA + CB
N_PAIRS = B_PER_W // PAIR
REM = B_PER_W - N_PAIRS * PAIR

_mesh = plsc.VectorSubcoreMesh(core_axis_name="c", subcore_axis_name="s")


@functools.partial(
    pl.kernel,
    mesh=_mesh,
    out_type=jax.ShapeDtypeStruct((N_TOK, HIDDEN), jnp.float32),
    scratch_types=[
        pltpu.VMEM((B_PER_W,), jnp.int32),
        pltpu.VMEM((CA, HIDDEN), jnp.float32),
        pltpu.VMEM((CB, HIDDEN), jnp.float32),
        pltpu.SemaphoreType.DMA((2,)),
        pltpu.SemaphoreType.DMA((2,)),
    ],
)
def _embed_sc(ids_hbm, tab_hbm, out_hbm, idx_v, buf_a, buf_b, sem_g, sem_s):
    wid = lax.axis_index("s") * NC + lax.axis_index("c")
    base = wid * B_PER_W
    pltpu.sync_copy(ids_hbm.at[pl.ds(base, B_PER_W)], idx_v)

    def gather(off, buf, sem):
        n = buf.shape[0]
        return pltpu.make_async_copy(
            tab_hbm.at[idx_v.at[pl.ds(off, n)]], buf, sem
        )

    def scatter(off, buf, sem):
        n = buf.shape[0]
        return pltpu.make_async_copy(
            buf, out_hbm.at[pl.ds(base + off, n)], sem
        )

    ga = lambda off: gather(off, buf_a, sem_g.at[0])
    gb = lambda off: gather(off, buf_b, sem_g.at[1])
    sa = lambda off: scatter(off, buf_a, sem_s.at[0])
    sb = lambda off: scatter(off, buf_b, sem_s.at[1])

    ga(0).start()
    gb(CA).start()

    def step(k, carry):
        off = k * PAIR
        ga(off).wait()
        sa(off).start()
        gb(off + CA).wait()
        sb(off + CA).start()
        nxt = off + PAIR

        @pl.when(nxt < B_PER_W)
        def _():
            sa(off).wait()
            ga(nxt).start()

        @pl.when(nxt + CA < B_PER_W)
        def _():
            sb(off + CA).wait()
            gb(nxt + CA).start()

        return carry

    lax.fori_loop(0, N_PAIRS, step, 0)

    last = N_PAIRS * PAIR
    ga(last).wait()
    sa(last).start()

    sb(last - CB).wait()
    sa(last).wait()


def kernel(input_ids, word_embeddings):
    ids = input_ids.reshape(-1).astype(jnp.int32)
    out = _embed_sc(ids, word_embeddings)
    return out.reshape(input_ids.shape + (word_embeddings.shape[1],))


# speedup vs baseline: 1.0204x; 1.0204x over previous
import functools

import jax
import jax.numpy as jnp
from jax import lax
from jax.experimental import pallas as pl
from jax.experimental.pallas import tpu as pltpu
from jax.experimental.pallas import tpu_sc as plsc

VOCAB = 168960
HIDDEN = 4096
N_TOK = 4 * 8192

NC = 2
NS = 16
NW = NC * NS
B_PER_W = N_TOK // NW
CHUNK = 8
N_CHUNKS = B_PER_W // CHUNK

_mesh = plsc.VectorSubcoreMesh(core_axis_name="c", subcore_axis_name="s")


NBUF = 3
TAIL = N_CHUNKS % NBUF
MAIN = N_CHUNKS - TAIL


@functools.partial(
    pl.kernel,
    mesh=_mesh,
    out_type=jax.ShapeDtypeStruct((N_TOK, HIDDEN), jnp.float32),
    scratch_types=[
        pltpu.VMEM((B_PER_W,), jnp.int32),
        pltpu.VMEM((NBUF, CHUNK, HIDDEN), jnp.float32),
        pltpu.SemaphoreType.DMA((NBUF,)),
        pltpu.SemaphoreType.DMA((NBUF,)),
    ],
)
def _embed_sc(ids_hbm, tab_hbm, out_hbm, idx_v, rows_v, sem_g, sem_s):
    wid = lax.axis_index("s") * NC + lax.axis_index("c")
    base = wid * B_PER_W
    pltpu.sync_copy(ids_hbm.at[pl.ds(base, B_PER_W)], idx_v)

    def gather(g, b):
        return pltpu.make_async_copy(
            tab_hbm.at[idx_v.at[pl.ds(g * CHUNK, CHUNK)]],
            rows_v.at[b],
            sem_g.at[b],
        )

    def scatter(g, b):
        return pltpu.make_async_copy(
            rows_v.at[b],
            out_hbm.at[pl.ds(base + g * CHUNK, CHUNK)],
            sem_s.at[b],
        )

    for b in range(NBUF):
        gather(b, b).start()

    def step(g0, carry):
        for b in range(NBUF):
            g = g0 * NBUF + b
            gather(g, b).wait()
            scatter(g, b).start()
            nxt = g + NBUF

            @pl.when(nxt < N_CHUNKS)
            def _():
                scatter(g, b).wait()
                gather(nxt, b).start()

        return carry

    lax.fori_loop(0, MAIN // NBUF, step, 0)

    for b in range(TAIL):
        g = MAIN + b
        gather(g, b).wait()
        scatter(g, b).start()

    for g in range(N_CHUNKS - NBUF, N_CHUNKS):
        scatter(g, g % NBUF).wait()


def kernel(input_ids, word_embeddings):
    ids = input_ids.reshape(-1).astype(jnp.int32)
    out = _embed_sc(ids, word_embeddings)
    return out.reshape(input_ids.shape + (word_embeddings.shape[1],))
